# Initial kernel scaffold; baseline (speedup 1.0000x reference)
import jax
import jax.numpy as jnp
from jax.experimental import pallas as pl


def kernel(labels, cuda):
    raise NotImplementedError("write your pallas kernel here")


# speedup vs baseline: 2.1267x; 2.1267x over previous
import jax
import jax.numpy as jnp
from jax.experimental import pallas as pl

_EPS = 1e-09
_NUM_CLASSES = 10
_BLOCK = 2048


def _onehot_kernel(labels_ref, mu_ref, std_ref):
    lab = labels_ref[...]
    classes = jax.lax.broadcasted_iota(jnp.int32, (_BLOCK, _NUM_CLASSES), 1)
    onehot = lab == classes
    mu_ref[...] = jnp.where(onehot, jnp.float32(1.0 - _EPS), jnp.float32(_EPS))
    std_ref[...] = jnp.full((_BLOCK, _NUM_CLASSES), _EPS, jnp.float32)


def kernel(labels, cuda):
    n = labels.shape[0]
    lab2 = labels.reshape(n, 1)
    grid = (n // _BLOCK,)
    mu, std = pl.pallas_call(
        _onehot_kernel,
        grid=grid,
        in_specs=[pl.BlockSpec((_BLOCK, 1), lambda i: (i, 0))],
        out_specs=[
            pl.BlockSpec((_BLOCK, _NUM_CLASSES), lambda i: (i, 0)),
            pl.BlockSpec((_BLOCK, _NUM_CLASSES), lambda i: (i, 0)),
        ],
        out_shape=[
            jax.ShapeDtypeStruct((n, _NUM_CLASSES), jnp.float32),
            jax.ShapeDtypeStruct((n, _NUM_CLASSES), jnp.float32),
        ],
    )(lab2)
    return mu[None, :, :], std[None, :, :]
